# baseline (speedup 1.0000x reference)
import jax
import jax.numpy as jnp
from jax.experimental import pallas as pl
from jax.experimental.pallas import tpu as pltpu

N = 50000
EMB = 512
G = 4096
TASKS = 128

BN = 1000
NBLK = N // BN
WS = 256
ROWS = G + WS


def _pool_body(meta_ref, batch_ref, x_ref, gw_ref, gb_ref,
               out_ref, acc_ref, dacc_ref, m_ref, xbuf_ref, sem_ref):
    i = pl.program_id(0)

    def x_copy(blk, slot):
        return pltpu.make_async_copy(
            x_ref.at[pl.ds(blk * BN, BN), :], xbuf_ref.at[slot],
            sem_ref.at[slot])

    @pl.when(i == 0)
    def _init():
        x_copy(0, 0).start()
        acc_ref[...] = jnp.zeros_like(acc_ref)
        dacc_ref[...] = jnp.zeros_like(dacc_ref)
        m_ref[0, 0] = -jnp.inf

    @pl.when(i + 1 < NBLK)
    def _prefetch():
        x_copy(i + 1, (i + 1) % 2).start()

    x_copy(i, i % 2).wait()
    xb = xbuf_ref[i % 2].astype(jnp.bfloat16)
    g = jax.lax.dot_general(
        gw_ref[...].astype(jnp.bfloat16), xb,
        dimension_numbers=(((0,), (1,)), ((), ())),
        preferred_element_type=jnp.float32)
    g = g + gb_ref[0, 0]
    m_b = jnp.max(g)
    m_old = m_ref[0, 0]
    m_new = jnp.maximum(m_old, m_b)

    @pl.when(m_b > m_old)
    def _rescale():
        scale = jnp.exp(m_old - m_new)
        acc_ref[...] = acc_ref[...] * scale
        dacc_ref[...] = dacc_ref[...] * scale
        m_ref[0, 0] = m_new

    e_row = jnp.exp(g - m_new)
    ones_col = jnp.ones((BN, 1), dtype=jnp.bfloat16)

    batch_row = batch_ref[0]
    base8 = meta_ref[0, 0, 0]
    nwin = meta_ref[0, 0, 1]
    row_ids = jax.lax.broadcasted_iota(jnp.int32, (WS, 1), 0)

    def win_step(j, _):
        start = pl.multiple_of(base8 + j * WS, 8)
        eq = row_ids + start == batch_row
        oh_e = jnp.where(eq, jnp.broadcast_to(e_row, (WS, BN)),
                         0.0).astype(jnp.bfloat16)
        contrib = jnp.dot(oh_e, xb, preferred_element_type=jnp.float32)
        dcontrib = jnp.dot(oh_e, ones_col, preferred_element_type=jnp.float32)
        acc_ref[pl.ds(start, WS), :] = acc_ref[pl.ds(start, WS), :] + contrib
        dacc_ref[pl.ds(start, WS), :] = (
            dacc_ref[pl.ds(start, WS), :] + dcontrib)
        return 0

    win_step(0, 0)
    jax.lax.fori_loop(1, nwin, win_step, 0)

    @pl.when(i == NBLK - 1)
    def _finish():
        out_ref[...] = acc_ref[:G, :] / (dacc_ref[:G, :] + 1e-16)


def _mlp_body(p_ref, w1_ref, b1_ref, w2_ref, b2_ref, w3_ref, b3_ref, o_ref):
    h = jnp.dot(p_ref[...], w1_ref[...], preferred_element_type=jnp.float32)
    h = jnp.maximum(h + b1_ref[...], 0.0)
    h = jnp.dot(h, w2_ref[...], preferred_element_type=jnp.float32)
    h = jnp.maximum(h + b2_ref[...], 0.0)
    o = jnp.dot(h, w3_ref[...], preferred_element_type=jnp.float32)
    o_ref[...] = o + b3_ref[...]


def kernel(node_representation, batch, gate_W, gate_b, W1, b1, W2, b2, W3, b3):
    batch = batch.astype(jnp.int32)
    batch3 = batch.reshape(NBLK, 1, BN)
    firsts = batch3[:, 0, 0]
    lasts = batch3[:, 0, BN - 1]
    base8 = (firsts // 8) * 8
    nwin = (lasts - base8) // WS + 1
    meta = jnp.stack([base8, nwin], axis=1).reshape(NBLK, 1, 2)

    pooled = pl.pallas_call(
        _pool_body,
        grid=(NBLK,),
        in_specs=[
            pl.BlockSpec((1, 1, 2), lambda i: (i, 0, 0),
                         memory_space=pltpu.SMEM),
            pl.BlockSpec((1, 1, BN), lambda i: (i, 0, 0)),
            pl.BlockSpec(memory_space=pltpu.HBM),
            pl.BlockSpec((EMB, 1), lambda i: (0, 0)),
            pl.BlockSpec((1, 1), lambda i: (0, 0), memory_space=pltpu.SMEM),
        ],
        out_specs=pl.BlockSpec((G, EMB), lambda i: (0, 0)),
        out_shape=jax.ShapeDtypeStruct((G, EMB), jnp.float32),
        scratch_shapes=[
            pltpu.VMEM((ROWS, EMB), jnp.float32),
            pltpu.VMEM((ROWS, 1), jnp.float32),
            pltpu.SMEM((1, 1), jnp.float32),
            pltpu.VMEM((2, BN, EMB), jnp.float32),
            pltpu.SemaphoreType.DMA((2,)),
        ],
        compiler_params=pltpu.CompilerParams(
            dimension_semantics=("arbitrary",)),
    )(meta, batch3, node_representation, gate_W,
      gate_b.reshape(1, 1))

    BG = 512
    logits = pl.pallas_call(
        _mlp_body,
        grid=(G // BG,),
        in_specs=[
            pl.BlockSpec((BG, EMB), lambda i: (i, 0)),
            pl.BlockSpec((EMB, EMB), lambda i: (0, 0)),
            pl.BlockSpec((1, EMB), lambda i: (0, 0)),
            pl.BlockSpec((EMB, EMB), lambda i: (0, 0)),
            pl.BlockSpec((1, EMB), lambda i: (0, 0)),
            pl.BlockSpec((EMB, TASKS), lambda i: (0, 0)),
            pl.BlockSpec((1, TASKS), lambda i: (0, 0)),
        ],
        out_specs=pl.BlockSpec((BG, TASKS), lambda i: (i, 0)),
        out_shape=jax.ShapeDtypeStruct((G, TASKS), jnp.float32),
    )(pooled, W1, b1.reshape(1, EMB), W2, b2.reshape(1, EMB),
      W3, b3.reshape(1, TASKS))

    return logits


# speedup vs baseline: 1.3103x; 1.3103x over previous
import jax
import jax.numpy as jnp
from jax.experimental import pallas as pl
from jax.experimental.pallas import tpu as pltpu

N = 50000
EMB = 512
G = 4096
TASKS = 128

BN = 2000
NBLK = N // BN
WS = 256
ROWS = G + WS


BG = 512
NMLP = G // BG


def _fused_body(meta_ref, batch_ref, x_ref, gw_ref, gb_ref,
                w1_ref, b1_ref, w2_ref, b2_ref, w3_ref, b3_ref,
                out_ref, acc_ref, dacc_ref, m_ref, xbuf_ref, sem_ref):
    i = pl.program_id(0)

    def x_copy(blk, slot):
        return pltpu.make_async_copy(
            x_ref.at[pl.ds(blk * BN, BN), :], xbuf_ref.at[slot],
            sem_ref.at[slot])

    @pl.when(i == 0)
    def _init():
        x_copy(0, 0).start()
        acc_ref[...] = jnp.zeros_like(acc_ref)
        dacc_ref[...] = jnp.zeros_like(dacc_ref)
        m_ref[0, 0] = -jnp.inf

    @pl.when(i + 1 < NBLK)
    def _prefetch():
        x_copy(i + 1, (i + 1) % 2).start()

    @pl.when(i < NBLK)
    def _pool_step():
        x_copy(i, i % 2).wait()
        xb = xbuf_ref[i % 2].astype(jnp.bfloat16)
        g = jax.lax.dot_general(
            gw_ref[...].astype(jnp.bfloat16), xb,
            dimension_numbers=(((0,), (1,)), ((), ())),
            preferred_element_type=jnp.float32)
        g = g + gb_ref[0, 0]
        m_b = jnp.max(g)
        m_old = m_ref[0, 0]
        m_new = jnp.maximum(m_old, m_b)

        @pl.when(m_b > m_old)
        def _rescale():
            scale = jnp.exp(m_old - m_new)
            acc_ref[...] = acc_ref[...] * scale
            dacc_ref[...] = dacc_ref[...] * scale
            m_ref[0, 0] = m_new

        e_row = jnp.exp(g - m_new)
        ones_col = jnp.ones((BN, 1), dtype=jnp.bfloat16)

        batch_row = batch_ref[0]
        base8 = meta_ref[0, 0, 0]
        nwin = meta_ref[0, 0, 1]
        row_ids = jax.lax.broadcasted_iota(jnp.int32, (WS, 1), 0)

        def win_step(j, _):
            start = pl.multiple_of(base8 + j * WS, 8)
            eq = row_ids + start == batch_row
            oh_e = jnp.where(eq, jnp.broadcast_to(e_row, (WS, BN)),
                             0.0).astype(jnp.bfloat16)
            contrib = jnp.dot(oh_e, xb, preferred_element_type=jnp.float32)
            dcontrib = jnp.dot(oh_e, ones_col,
                               preferred_element_type=jnp.float32)
            acc_ref[pl.ds(start, WS), :] = (
                acc_ref[pl.ds(start, WS), :] + contrib)
            dacc_ref[pl.ds(start, WS), :] = (
                dacc_ref[pl.ds(start, WS), :] + dcontrib)
            return 0

        win_step(0, 0)
        jax.lax.fori_loop(1, nwin, win_step, 0)

    @pl.when(i >= NBLK)
    def _mlp_step():
        off = pl.multiple_of((i - NBLK) * BG, 8)
        p = (acc_ref[pl.ds(off, BG), :]
             / (dacc_ref[pl.ds(off, BG), :] + 1e-16))
        h = jnp.dot(p, w1_ref[...], preferred_element_type=jnp.float32)
        h = jnp.maximum(h + b1_ref[...], 0.0)
        h = jnp.dot(h, w2_ref[...], preferred_element_type=jnp.float32)
        h = jnp.maximum(h + b2_ref[...], 0.0)
        o = jnp.dot(h, w3_ref[...], preferred_element_type=jnp.float32)
        out_ref[...] = o + b3_ref[...]


def kernel(node_representation, batch, gate_W, gate_b, W1, b1, W2, b2, W3, b3):
    batch = batch.astype(jnp.int32)
    batch3 = batch.reshape(NBLK, 1, BN)
    firsts = batch3[:, 0, 0]
    lasts = batch3[:, 0, BN - 1]
    base8 = (firsts // 8) * 8
    nwin = (lasts - base8) // WS + 1
    meta = jnp.stack([base8, nwin], axis=1).reshape(NBLK, 1, 2)

    logits = pl.pallas_call(
        _fused_body,
        grid=(NBLK + NMLP,),
        in_specs=[
            pl.BlockSpec((1, 1, 2), lambda i: (jnp.minimum(i, NBLK - 1), 0, 0),
                         memory_space=pltpu.SMEM),
            pl.BlockSpec((1, 1, BN), lambda i: (jnp.minimum(i, NBLK - 1), 0, 0)),
            pl.BlockSpec(memory_space=pltpu.HBM),
            pl.BlockSpec((EMB, 1), lambda i: (0, 0)),
            pl.BlockSpec((1, 1), lambda i: (0, 0), memory_space=pltpu.SMEM),
            pl.BlockSpec((EMB, EMB), lambda i: (0, 0)),
            pl.BlockSpec((1, EMB), lambda i: (0, 0)),
            pl.BlockSpec((EMB, EMB), lambda i: (0, 0)),
            pl.BlockSpec((1, EMB), lambda i: (0, 0)),
            pl.BlockSpec((EMB, TASKS), lambda i: (0, 0)),
            pl.BlockSpec((1, TASKS), lambda i: (0, 0)),
        ],
        out_specs=pl.BlockSpec(
            (BG, TASKS), lambda i: (jnp.maximum(i - NBLK, 0), 0)),
        out_shape=jax.ShapeDtypeStruct((G, TASKS), jnp.float32),
        scratch_shapes=[
            pltpu.VMEM((ROWS, EMB), jnp.float32),
            pltpu.VMEM((ROWS, 1), jnp.float32),
            pltpu.SMEM((1, 1), jnp.float32),
            pltpu.VMEM((2, BN, EMB), jnp.float32),
            pltpu.SemaphoreType.DMA((2,)),
        ],
        compiler_params=pltpu.CompilerParams(
            dimension_semantics=("arbitrary",)),
    )(meta, batch3, node_representation, gate_W, gate_b.reshape(1, 1),
      W1, b1.reshape(1, EMB), W2, b2.reshape(1, EMB),
      W3, b3.reshape(1, TASKS))

    return logits


# speedup vs baseline: 1.3749x; 1.0493x over previous
import jax
import jax.numpy as jnp
from jax.experimental import pallas as pl
from jax.experimental.pallas import tpu as pltpu

N = 50000
EMB = 512
G = 4096
TASKS = 128

BN = 2000
NBLK = N // BN
WS = 256
ROWS = G + WS


BG = 512
NMLP = G // BG


def _fused_body(meta_ref, batch_ref, x_ref, gw_ref, gb_ref,
                w1_ref, b1_ref, w2_ref, b2_ref, w3_ref, b3_ref,
                out_ref, acc_ref, dacc_ref, m_ref, xbuf_ref, sem_ref):
    i = pl.program_id(0)

    def x_copy(blk, slot):
        return pltpu.make_async_copy(
            x_ref.at[pl.ds(blk * BN, BN), :], xbuf_ref.at[slot],
            sem_ref.at[slot])

    @pl.when(i == 0)
    def _init():
        x_copy(0, 0).start()
        acc_ref[...] = jnp.zeros_like(acc_ref)
        dacc_ref[...] = jnp.zeros_like(dacc_ref)
        m_ref[0, 0] = -jnp.inf

    @pl.when(i + 1 < NBLK)
    def _prefetch():
        x_copy(i + 1, (i + 1) % 2).start()

    @pl.when(i < NBLK)
    def _pool_step():
        x_copy(i, i % 2).wait()
        xb = xbuf_ref[i % 2].astype(jnp.bfloat16)
        g = jax.lax.dot_general(
            gw_ref[...].astype(jnp.bfloat16), xb,
            dimension_numbers=(((0,), (1,)), ((), ())),
            preferred_element_type=jnp.float32)
        g = g + gb_ref[0, 0]
        m_b = jnp.max(g)
        m_old = m_ref[0, 0]
        m_new = jnp.maximum(m_old, m_b)

        @pl.when(m_b > m_old)
        def _rescale():
            scale = jnp.exp(m_old - m_new)
            acc_ref[...] = acc_ref[...] * scale
            dacc_ref[...] = dacc_ref[...] * scale
            m_ref[0, 0] = m_new

        e_row = jnp.exp(g - m_new)
        ones_col = jnp.ones((BN, 1), dtype=jnp.bfloat16)

        batch_row = batch_ref[0]
        base8 = meta_ref[0, 0, 0]
        nwin = meta_ref[0, 0, 1]
        row_ids = jax.lax.broadcasted_iota(jnp.int32, (WS, 1), 0)

        def win_step(j, _):
            start = pl.multiple_of(base8 + j * WS, 8)
            eq = row_ids + start == batch_row
            oh_e = jnp.where(eq, jnp.broadcast_to(e_row, (WS, BN)),
                             0.0).astype(jnp.bfloat16)
            contrib = jnp.dot(oh_e, xb, preferred_element_type=jnp.float32)
            dcontrib = jnp.sum(oh_e.astype(jnp.float32), axis=1,
                               keepdims=True)
            acc_ref[pl.ds(start, WS), :] = (
                acc_ref[pl.ds(start, WS), :] + contrib)
            dacc_ref[pl.ds(start, WS), :] = (
                dacc_ref[pl.ds(start, WS), :] + dcontrib)
            return 0

        win_step(0, 0)
        jax.lax.fori_loop(1, nwin, win_step, 0)

    @pl.when(i >= NBLK)
    def _mlp_step():
        off = pl.multiple_of((i - NBLK) * BG, 8)
        p = (acc_ref[pl.ds(off, BG), :]
             / (dacc_ref[pl.ds(off, BG), :] + 1e-16))
        h = jnp.dot(p, w1_ref[...], preferred_element_type=jnp.float32)
        h = jnp.maximum(h + b1_ref[...], 0.0)
        h = jnp.dot(h, w2_ref[...], preferred_element_type=jnp.float32)
        h = jnp.maximum(h + b2_ref[...], 0.0)
        o = jnp.dot(h, w3_ref[...], preferred_element_type=jnp.float32)
        out_ref[...] = o + b3_ref[...]


def kernel(node_representation, batch, gate_W, gate_b, W1, b1, W2, b2, W3, b3):
    batch = batch.astype(jnp.int32)
    batch3 = batch.reshape(NBLK, 1, BN)
    firsts = batch3[:, 0, 0]
    lasts = batch3[:, 0, BN - 1]
    base8 = (firsts // 8) * 8
    nwin = (lasts - base8) // WS + 1
    meta = jnp.stack([base8, nwin], axis=1).reshape(NBLK, 1, 2)

    logits = pl.pallas_call(
        _fused_body,
        grid=(NBLK + NMLP,),
        in_specs=[
            pl.BlockSpec((1, 1, 2), lambda i: (jnp.minimum(i, NBLK - 1), 0, 0),
                         memory_space=pltpu.SMEM),
            pl.BlockSpec((1, 1, BN), lambda i: (jnp.minimum(i, NBLK - 1), 0, 0)),
            pl.BlockSpec(memory_space=pltpu.HBM),
            pl.BlockSpec((EMB, 1), lambda i: (0, 0)),
            pl.BlockSpec((1, 1), lambda i: (0, 0), memory_space=pltpu.SMEM),
            pl.BlockSpec((EMB, EMB), lambda i: (0, 0)),
            pl.BlockSpec((1, EMB), lambda i: (0, 0)),
            pl.BlockSpec((EMB, EMB), lambda i: (0, 0)),
            pl.BlockSpec((1, EMB), lambda i: (0, 0)),
            pl.BlockSpec((EMB, TASKS), lambda i: (0, 0)),
            pl.BlockSpec((1, TASKS), lambda i: (0, 0)),
        ],
        out_specs=pl.BlockSpec(
            (BG, TASKS), lambda i: (jnp.maximum(i - NBLK, 0), 0)),
        out_shape=jax.ShapeDtypeStruct((G, TASKS), jnp.float32),
        scratch_shapes=[
            pltpu.VMEM((ROWS, EMB), jnp.float32),
            pltpu.VMEM((ROWS, 1), jnp.float32),
            pltpu.SMEM((1, 1), jnp.float32),
            pltpu.VMEM((2, BN, EMB), jnp.float32),
            pltpu.SemaphoreType.DMA((2,)),
        ],
        compiler_params=pltpu.CompilerParams(
            dimension_semantics=("arbitrary",)),
    )(meta, batch3, node_representation, gate_W, gate_b.reshape(1, 1),
      W1, b1.reshape(1, EMB), W2, b2.reshape(1, EMB),
      W3, b3.reshape(1, TASKS))

    return logits
